# baseline (speedup 1.0000x reference)
import functools

import jax
import jax.numpy as jnp
from jax import lax
from jax.experimental import pallas as pl
from jax.experimental.pallas import tpu as pltpu
from jax.experimental.pallas import tpu_sc as plsc

N_CLS = 1000
N_CTX = 16
D = 512
SEQ = 77
SUF = SEQ - 1 - N_CTX
SUF_MAIN = 56
SUF_TAIL = 8
SUF_PAD = SUF_MAIN + SUF_TAIL
B = 1024
SEQ_PAD = 128

NC, NS = 2, 16
NW = NC * NS
SPLIT = 512
NCHUNK = 2
BC = SPLIT // NCHUNK
BPW = BC // NW
CTX_CH = 2
TAIL_CH = 4


def _sc_body(cls_w, cls_p, cls_c, cls_t,
             pre_n, ctx_n, suf_n, pre_p, ctx_p, suf_p,
             tail_n, tail_p, tok_n, tok_p,
             out_pre, out_ctx, out_suf, out_tok,
             idx_v, idxp_v, idxc_v, idxt_v,
             pbuf, tbuf, cbuf0, cbuf1, sbuf0, sbuf1, lbuf,
             gsem_p, ssem_p, gsem_t, ssem_t, gsem_l, ssem_l,
             gsem_c0, gsem_c1, ssem_c0, ssem_c1,
             gsem_s0, gsem_s1, ssem_s0, ssem_s1):
    wid = lax.axis_index("s") * NC + lax.axis_index("c")
    base = wid * BPW

    pltpu.sync_copy(cls_w.at[wid], idx_v)
    pltpu.sync_copy(cls_p.at[wid], idxp_v)
    pltpu.sync_copy(cls_c.at[wid], idxc_v)
    pltpu.sync_copy(cls_t.at[wid], idxt_v)

    slots = {
        "s0": (sbuf0, gsem_s0, ssem_s0),
        "s1": (sbuf1, gsem_s1, ssem_s1),
        "c0": (cbuf0, gsem_c0, ssem_c0),
        "c1": (cbuf1, gsem_c1, ssem_c1),
        "p": (pbuf, gsem_p, ssem_p),
        "t": (tbuf, gsem_t, ssem_t),
        "l": (lbuf, gsem_l, ssem_l),
    }

    task_list = []
    halves = ((pre_n, ctx_n, suf_n, tail_n, tok_n, 0),
              (pre_p, ctx_p, suf_p, tail_p, tok_p, 1))
    for pol, (pre_t, ctx_t, suf_t, tail_t, tok_t, _) in enumerate(halves):
        ob = pol * BC + base
        for j in range(BPW):
            idx = idxc_v.at[pol * (BPW // CTX_CH) + j // CTX_CH,
                            pl.ds(j % CTX_CH, 1)]
            dst = out_suf.at[pl.ds(ob + j, 1), pl.ds(0, SUF_MAIN), :]
            task_list.append((suf_t.at[idx, pl.ds(0, SUF_MAIN), :], dst,
                              "s0" if j % 2 == 0 else "s1"))
            if j % CTX_CH == 0:
                jc = j // CTX_CH
                idxc = idxc_v.at[pol * (BPW // CTX_CH) + jc]
                cdst = out_ctx.at[pl.ds(ob + jc * CTX_CH, CTX_CH)]
                task_list.append((ctx_t.at[idxc], cdst,
                                  "c0" if jc % 2 == 0 else "c1"))
            if j % TAIL_CH == 0:
                jt = j // TAIL_CH
                idxt = idxt_v.at[pol * (BPW // TAIL_CH) + jt]
                ldst = out_suf.at[pl.ds(ob + jt * TAIL_CH, TAIL_CH),
                                  pl.ds(SUF_MAIN, SUF_TAIL), :]
                task_list.append((tail_t.at[idxt], ldst, "l"))
            if j == 1:
                task_list.append((pre_t.at[idxp_v.at[pol]],
                                  out_pre.at[pl.ds(ob, BPW)], "p"))
            if j == 3:
                task_list.append((tok_t.at[idxp_v.at[pol]],
                                  out_tok.at[pl.ds(ob, BPW), :], "t"))

    last_scatter = {}
    prev = None
    for src, dst, slot in task_list:
        buf, gsem, _ = slots[slot]
        if slot in last_scatter:
            last_scatter.pop(slot).wait()
        g = pltpu.async_copy(src, buf, gsem)
        if prev is not None:
            pg, pdst, pslot = prev
            pbuf_, _, pssem = slots[pslot]
            pg.wait()
            last_scatter[pslot] = pltpu.async_copy(pbuf_, pdst, pssem)
        prev = (g, dst, slot)
    pg, pdst, pslot = prev
    pbuf_, _, pssem = slots[pslot]
    pg.wait()
    last_scatter[pslot] = pltpu.async_copy(pbuf_, pdst, pssem)
    for s in last_scatter.values():
        s.wait()


@functools.lru_cache(maxsize=None)
def _make_call():
    mesh = plsc.VectorSubcoreMesh(core_axis_name="c", subcore_axis_name="s",
                                  num_cores=NC, num_subcores=NS)
    return pl.kernel(
        _sc_body,
        out_type=(
            jax.ShapeDtypeStruct((2 * BC, D), jnp.float32),
            jax.ShapeDtypeStruct((2 * BC, N_CTX, D), jnp.float32),
            jax.ShapeDtypeStruct((2 * BC, SUF_PAD, D), jnp.float32),
            jax.ShapeDtypeStruct((2 * BC, SEQ_PAD), jnp.int32),
        ),
        mesh=mesh,
        scratch_types=[
            pltpu.VMEM((BPW,), jnp.int32),
            pltpu.VMEM((2, BPW), jnp.int32),
            pltpu.VMEM((2 * BPW // CTX_CH, CTX_CH), jnp.int32),
            pltpu.VMEM((2 * BPW // TAIL_CH, TAIL_CH), jnp.int32),
            pltpu.VMEM((BPW, D), jnp.float32),
            pltpu.VMEM((BPW, SEQ_PAD), jnp.int32),
            pltpu.VMEM((CTX_CH, N_CTX, D), jnp.float32),
            pltpu.VMEM((CTX_CH, N_CTX, D), jnp.float32),
            pltpu.VMEM((1, SUF_MAIN, D), jnp.float32),
            pltpu.VMEM((1, SUF_MAIN, D), jnp.float32),
            pltpu.VMEM((TAIL_CH, SUF_TAIL, D), jnp.float32),
        ] + [pltpu.SemaphoreType.DMA] * 14,
    )


def _tc_gather_body(cls_ref, pre, ctx, suf, out_ref):
    del cls_ref
    out_ref[0, 0, :] = pre[0, 0, :]
    out_ref[0, 1:1 + N_CTX, :] = ctx[0]
    out_ref[0, 1 + N_CTX:, :] = suf[0]


def _tc_gather(cls_hi, pre_t, ctx_t, suf_t, row0, prompts):
    def tbl3(block):
        return pl.BlockSpec(block, lambda e, cls: (cls[e], 0, 0))

    seg_specs = [tbl3((1, 1, D)), tbl3((1, N_CTX, D)), tbl3((1, SUF, D))]
    out_specs = pl.BlockSpec((1, SEQ, D),
                             lambda e, cls: (row0 + SPLIT + e, 0, 0))
    out_shape = jax.ShapeDtypeStruct((2 * B, SEQ, D), jnp.float32)
    if prompts is None:
        grid_spec = pltpu.PrefetchScalarGridSpec(
            num_scalar_prefetch=1, grid=(B - SPLIT,),
            in_specs=seg_specs, out_specs=out_specs)
        return pl.pallas_call(
            _tc_gather_body, grid_spec=grid_spec, out_shape=out_shape,
        )(cls_hi, pre_t, ctx_t, suf_t)

    def body(cls_ref, buf, pre, ctx, suf, out_ref):
        _tc_gather_body(cls_ref, pre, ctx, suf, out_ref)

    grid_spec = pltpu.PrefetchScalarGridSpec(
        num_scalar_prefetch=1, grid=(B - SPLIT,),
        in_specs=[pl.BlockSpec(memory_space=pl.ANY)] + seg_specs,
        out_specs=out_specs)
    return pl.pallas_call(
        body, grid_spec=grid_spec, out_shape=out_shape,
        input_output_aliases={1: 0},
    )(cls_hi, prompts, pre_t, ctx_t, suf_t)


_TOK_BLK = 8


def _tc_tok_gather(cls_hi, tok3, row0, tok_out):
    def tok_spec(k):
        return pl.BlockSpec((1, 1, SEQ_PAD),
                            lambda s, cls, k=k: (cls[_TOK_BLK * s + k], 0, 0))

    seg_specs = [tok_spec(k) for k in range(_TOK_BLK)]
    out_specs = pl.BlockSpec(
        (_TOK_BLK, SEQ_PAD),
        lambda s, cls: ((row0 + SPLIT) // _TOK_BLK + s, 0))
    out_shape = jax.ShapeDtypeStruct((2 * B, SEQ_PAD), jnp.int32)

    def body_create(cls_ref, *refs):
        del cls_ref
        out_ref = refs[_TOK_BLK]
        for k in range(_TOK_BLK):
            out_ref[k, :] = refs[k][0, 0, :]

    if tok_out is None:
        grid_spec = pltpu.PrefetchScalarGridSpec(
            num_scalar_prefetch=1, grid=((B - SPLIT) // _TOK_BLK,),
            in_specs=seg_specs, out_specs=out_specs)
        return pl.pallas_call(
            body_create, grid_spec=grid_spec, out_shape=out_shape,
        )(cls_hi, *([tok3] * _TOK_BLK))

    def body_alias(cls_ref, buf, *refs):
        del cls_ref, buf
        out_ref = refs[_TOK_BLK]
        for k in range(_TOK_BLK):
            out_ref[k, :] = refs[k][0, 0, :]

    grid_spec = pltpu.PrefetchScalarGridSpec(
        num_scalar_prefetch=1, grid=((B - SPLIT) // _TOK_BLK,),
        in_specs=[pl.BlockSpec(memory_space=pl.ANY)] + seg_specs,
        out_specs=out_specs)
    return pl.pallas_call(
        body_alias, grid_spec=grid_spec, out_shape=out_shape,
        input_output_aliases={1: 0},
    )(cls_hi, tok_out, *([tok3] * _TOK_BLK))


def _tc_asm_body(*refs):
    pre_ref, ctx_ref, suf_ref, tokc_ref = refs[-6:-2]
    out_ref, otok_ref = refs[-2:]
    out_ref[:, 0, :] = pre_ref[...]
    out_ref[:, 1:1 + N_CTX, :] = ctx_ref[...]
    out_ref[:, 1 + N_CTX:SEQ, :] = suf_ref[:, :SUF, :]
    otok_ref[...] = tokc_ref[...]


def _tc_asm(i, prompts, tokenized, pre, ctx, suf, tok):
    blk = 32
    grid = (2, BC // blk)
    nb, cb = B // blk, BC // blk

    def seg_map(pol, s):
        return (pol * cb + s, 0)

    def seg3_map(pol, s):
        return (pol * cb + s, 0, 0)

    def out_map(pol, s):
        return (pol * nb + i * cb + s, 0, 0)

    def otok_map(pol, s):
        return (pol * nb + i * cb + s, 0)

    seg_specs = [
        pl.BlockSpec((blk, D), seg_map),
        pl.BlockSpec((blk, N_CTX, D), seg3_map),
        pl.BlockSpec((blk, SUF_PAD, D), seg3_map),
        pl.BlockSpec((blk, SEQ_PAD), seg_map),
    ]
    out_shape = (
        jax.ShapeDtypeStruct((2 * B, SEQ, D), jnp.float32),
        jax.ShapeDtypeStruct((2 * B, SEQ_PAD), jnp.int32),
    )
    out_specs = (
        pl.BlockSpec((blk, SEQ, D), out_map),
        pl.BlockSpec((blk, SEQ_PAD), otok_map),
    )
    return pl.pallas_call(
        _tc_asm_body, grid=grid,
        in_specs=[pl.BlockSpec(memory_space=pl.ANY),
                  pl.BlockSpec(memory_space=pl.ANY)] + seg_specs,
        out_specs=out_specs, out_shape=out_shape,
        input_output_aliases={0: 0, 1: 1},
    )(prompts, tokenized, pre, ctx, suf, tok)


@jax.jit
def _prompt_gather(cls_id, ctx_pos, ctx_neg, pre_pos2, suf_pos,
                   pre_neg2, suf_neg, tail_pos, tail_neg, tok_neg, tok_pos,
                   pre_pos3, pre_neg3):
    call = _make_call()
    cls_hi = cls_id[SPLIT:]
    prompts = _tc_gather(cls_hi, pre_neg3, ctx_neg, suf_neg, 0, None)
    prompts = _tc_gather(cls_hi, pre_pos3, ctx_pos, suf_pos, B, prompts)
    tok_neg3 = tok_neg.reshape(N_CLS, 1, SEQ_PAD)
    tok_pos3 = tok_pos.reshape(N_CLS, 1, SEQ_PAD)
    tokenized = _tc_tok_gather(cls_hi, tok_neg3, 0, None)
    tokenized = _tc_tok_gather(cls_hi, tok_pos3, B, tokenized)
    for i in range(NCHUNK):
        cls_c = lax.dynamic_slice_in_dim(cls_id, i * BC, BC)
        cls_w = cls_c.reshape(NW, BPW)
        cls2 = jnp.concatenate([cls_w] * 2, axis=1)
        pre, ctx, suf, tok = call(
            cls_w,
            cls2.reshape(NW, 2, BPW),
            cls2.reshape(NW, 2 * BPW // CTX_CH, CTX_CH),
            cls2.reshape(NW, 2 * BPW // TAIL_CH, TAIL_CH),
            pre_neg2, ctx_neg, suf_neg,
            pre_pos2, ctx_pos, suf_pos,
            tail_neg, tail_pos, tok_neg, tok_pos)
        prompts, tokenized = _tc_asm(i, prompts, tokenized,
                                     pre, ctx, suf, tok)
    return prompts, tokenized[:, :SEQ]


def kernel(cls_id, ctx_pos, ctx_neg, token_prefix_pos, token_suffix_pos,
           token_prefix_neg, token_suffix_neg, tokenized_prompts):
    n_cls = ctx_pos.shape[0]
    pad_tail = ((0, 0), (0, SUF_TAIL - (SUF - SUF_MAIN)), (0, 0))
    return _prompt_gather(
        cls_id, ctx_pos, ctx_neg,
        token_prefix_pos.reshape(n_cls, D),
        token_suffix_pos,
        token_prefix_neg.reshape(n_cls, D),
        token_suffix_neg,
        jnp.pad(token_suffix_pos[:, SUF_MAIN:, :], pad_tail),
        jnp.pad(token_suffix_neg[:, SUF_MAIN:, :], pad_tail),
        jnp.pad(tokenized_prompts[:n_cls], ((0, 0), (0, SEQ_PAD - SEQ))),
        jnp.pad(tokenized_prompts[n_cls:], ((0, 0), (0, SEQ_PAD - SEQ))),
        token_prefix_pos, token_prefix_neg,
    )


# speedup vs baseline: 1.3772x; 1.3772x over previous
import functools

import jax
import jax.numpy as jnp
from jax import lax
from jax.experimental import pallas as pl
from jax.experimental.pallas import tpu as pltpu
from jax.experimental.pallas import tpu_sc as plsc

N_CLS = 1000
N_CTX = 16
D = 512
SEQ = 77
SUF = SEQ - 1 - N_CTX
SUF_MAIN = 56
SUF_TAIL = 8
SUF_PAD = SUF_MAIN + SUF_TAIL
B = 1024
SEQ_PAD = 128

NC, NS = 2, 16
NW = NC * NS
SPLIT = 0
NCHUNK = 8
BC = B // NCHUNK
BPW = BC // NW
CTX_CH = 2
TAIL_CH = 4


def _sc_body(cls_w, cls_p, cls_c, cls_t,
             pre_n, ctx_n, suf_n, pre_p, ctx_p, suf_p,
             tail_n, tail_p, tok_n, tok_p,
             out_pre, out_ctx, out_suf, out_tok,
             idx_v, idxp_v, idxc_v, idxt_v,
             pbuf, tbuf, cbuf0, cbuf1, sbuf0, sbuf1, lbuf,
             gsem_p, ssem_p, gsem_t, ssem_t, gsem_l, ssem_l,
             gsem_c0, gsem_c1, ssem_c0, ssem_c1,
             gsem_s0, gsem_s1, ssem_s0, ssem_s1):
    wid = lax.axis_index("s") * NC + lax.axis_index("c")
    base = wid * BPW

    pltpu.sync_copy(cls_w.at[wid], idx_v)
    pltpu.sync_copy(cls_p.at[wid], idxp_v)
    pltpu.sync_copy(cls_c.at[wid], idxc_v)
    pltpu.sync_copy(cls_t.at[wid], idxt_v)

    slots = {
        "s0": (sbuf0, gsem_s0, ssem_s0),
        "s1": (sbuf1, gsem_s1, ssem_s1),
        "c0": (cbuf0, gsem_c0, ssem_c0),
        "c1": (cbuf1, gsem_c1, ssem_c1),
        "p": (pbuf, gsem_p, ssem_p),
        "t": (tbuf, gsem_t, ssem_t),
        "l": (lbuf, gsem_l, ssem_l),
    }

    task_list = []
    halves = ((pre_n, ctx_n, suf_n, tail_n, tok_n, 0),
              (pre_p, ctx_p, suf_p, tail_p, tok_p, 1))
    for pol, (pre_t, ctx_t, suf_t, tail_t, tok_t, _) in enumerate(halves):
        ob = pol * BC + base
        for j in range(BPW):
            idx = idxc_v.at[pol * (BPW // CTX_CH) + j // CTX_CH,
                            pl.ds(j % CTX_CH, 1)]
            dst = out_suf.at[pl.ds(ob + j, 1), pl.ds(0, SUF_MAIN), :]
            task_list.append((suf_t.at[idx, pl.ds(0, SUF_MAIN), :], dst,
                              "s0" if j % 2 == 0 else "s1"))
            if j % CTX_CH == 0:
                jc = j // CTX_CH
                idxc = idxc_v.at[pol * (BPW // CTX_CH) + jc]
                cdst = out_ctx.at[pl.ds(ob + jc * CTX_CH, CTX_CH)]
                task_list.append((ctx_t.at[idxc], cdst,
                                  "c0" if jc % 2 == 0 else "c1"))
            if j % TAIL_CH == 0:
                jt = j // TAIL_CH
                idxt = idxt_v.at[pol * (BPW // TAIL_CH) + jt]
                ldst = out_suf.at[pl.ds(ob + jt * TAIL_CH, TAIL_CH),
                                  pl.ds(SUF_MAIN, SUF_TAIL), :]
                task_list.append((tail_t.at[idxt], ldst, "l"))
            if j == 1:
                task_list.append((pre_t.at[idxp_v.at[pol]],
                                  out_pre.at[pl.ds(ob, BPW)], "p"))
            if j == 3:
                task_list.append((tok_t.at[idxp_v.at[pol]],
                                  out_tok.at[pl.ds(ob, BPW), :], "t"))

    last_scatter = {}
    prev = None
    for src, dst, slot in task_list:
        buf, gsem, _ = slots[slot]
        if slot in last_scatter:
            last_scatter.pop(slot).wait()
        g = pltpu.async_copy(src, buf, gsem)
        if prev is not None:
            pg, pdst, pslot = prev
            pbuf_, _, pssem = slots[pslot]
            pg.wait()
            last_scatter[pslot] = pltpu.async_copy(pbuf_, pdst, pssem)
        prev = (g, dst, slot)
    pg, pdst, pslot = prev
    pbuf_, _, pssem = slots[pslot]
    pg.wait()
    last_scatter[pslot] = pltpu.async_copy(pbuf_, pdst, pssem)
    for s in last_scatter.values():
        s.wait()


@functools.lru_cache(maxsize=None)
def _make_call():
    mesh = plsc.VectorSubcoreMesh(core_axis_name="c", subcore_axis_name="s",
                                  num_cores=NC, num_subcores=NS)
    return pl.kernel(
        _sc_body,
        out_type=(
            jax.ShapeDtypeStruct((2 * BC, D), jnp.float32),
            jax.ShapeDtypeStruct((2 * BC, N_CTX, D), jnp.float32),
            jax.ShapeDtypeStruct((2 * BC, SUF_PAD, D), jnp.float32),
            jax.ShapeDtypeStruct((2 * BC, SEQ_PAD), jnp.int32),
        ),
        mesh=mesh,
        scratch_types=[
            pltpu.VMEM((BPW,), jnp.int32),
            pltpu.VMEM((2, BPW), jnp.int32),
            pltpu.VMEM((2 * BPW // CTX_CH, CTX_CH), jnp.int32),
            pltpu.VMEM((2 * BPW // TAIL_CH, TAIL_CH), jnp.int32),
            pltpu.VMEM((BPW, D), jnp.float32),
            pltpu.VMEM((BPW, SEQ_PAD), jnp.int32),
            pltpu.VMEM((CTX_CH, N_CTX, D), jnp.float32),
            pltpu.VMEM((CTX_CH, N_CTX, D), jnp.float32),
            pltpu.VMEM((1, SUF_MAIN, D), jnp.float32),
            pltpu.VMEM((1, SUF_MAIN, D), jnp.float32),
            pltpu.VMEM((TAIL_CH, SUF_TAIL, D), jnp.float32),
        ] + [pltpu.SemaphoreType.DMA] * 14,
    )


def _tc_gather_body(cls_ref, pre, ctx, suf, out_ref):
    del cls_ref
    out_ref[0, 0, :] = pre[0, 0, :]
    out_ref[0, 1:1 + N_CTX, :] = ctx[0]
    out_ref[0, 1 + N_CTX:, :] = suf[0]


def _tc_gather(cls_hi, pre_t, ctx_t, suf_t, row0, prompts):
    def tbl3(block):
        return pl.BlockSpec(block, lambda e, cls: (cls[e], 0, 0))

    seg_specs = [tbl3((1, 1, D)), tbl3((1, N_CTX, D)), tbl3((1, SUF, D))]
    out_specs = pl.BlockSpec((1, SEQ, D),
                             lambda e, cls: (row0 + SPLIT + e, 0, 0))
    out_shape = jax.ShapeDtypeStruct((2 * B, SEQ, D), jnp.float32)
    if prompts is None:
        grid_spec = pltpu.PrefetchScalarGridSpec(
            num_scalar_prefetch=1, grid=(B - SPLIT,),
            in_specs=seg_specs, out_specs=out_specs)
        return pl.pallas_call(
            _tc_gather_body, grid_spec=grid_spec, out_shape=out_shape,
        )(cls_hi, pre_t, ctx_t, suf_t)

    def body(cls_ref, buf, pre, ctx, suf, out_ref):
        _tc_gather_body(cls_ref, pre, ctx, suf, out_ref)

    grid_spec = pltpu.PrefetchScalarGridSpec(
        num_scalar_prefetch=1, grid=(B - SPLIT,),
        in_specs=[pl.BlockSpec(memory_space=pl.ANY)] + seg_specs,
        out_specs=out_specs)
    return pl.pallas_call(
        body, grid_spec=grid_spec, out_shape=out_shape,
        input_output_aliases={1: 0},
    )(cls_hi, prompts, pre_t, ctx_t, suf_t)


_TOK_BLK = 8


def _tc_tok_gather(cls_hi, tok3, row0, tok_out):
    def tok_spec(k):
        return pl.BlockSpec((1, 1, SEQ_PAD),
                            lambda s, cls, k=k: (cls[_TOK_BLK * s + k], 0, 0))

    seg_specs = [tok_spec(k) for k in range(_TOK_BLK)]
    out_specs = pl.BlockSpec(
        (_TOK_BLK, SEQ_PAD),
        lambda s, cls: ((row0 + SPLIT) // _TOK_BLK + s, 0))
    out_shape = jax.ShapeDtypeStruct((2 * B, SEQ_PAD), jnp.int32)

    def body_create(cls_ref, *refs):
        del cls_ref
        out_ref = refs[_TOK_BLK]
        for k in range(_TOK_BLK):
            out_ref[k, :] = refs[k][0, 0, :]

    if tok_out is None:
        grid_spec = pltpu.PrefetchScalarGridSpec(
            num_scalar_prefetch=1, grid=((B - SPLIT) // _TOK_BLK,),
            in_specs=seg_specs, out_specs=out_specs)
        return pl.pallas_call(
            body_create, grid_spec=grid_spec, out_shape=out_shape,
        )(cls_hi, *([tok3] * _TOK_BLK))

    def body_alias(cls_ref, buf, *refs):
        del cls_ref, buf
        out_ref = refs[_TOK_BLK]
        for k in range(_TOK_BLK):
            out_ref[k, :] = refs[k][0, 0, :]

    grid_spec = pltpu.PrefetchScalarGridSpec(
        num_scalar_prefetch=1, grid=((B - SPLIT) // _TOK_BLK,),
        in_specs=[pl.BlockSpec(memory_space=pl.ANY)] + seg_specs,
        out_specs=out_specs)
    return pl.pallas_call(
        body_alias, grid_spec=grid_spec, out_shape=out_shape,
        input_output_aliases={1: 0},
    )(cls_hi, tok_out, *([tok3] * _TOK_BLK))


def _tc_asm_body(*refs):
    pre_ref, ctx_ref, suf_ref, tokc_ref = refs[-6:-2]
    out_ref, otok_ref = refs[-2:]
    out_ref[:, 0, :] = pre_ref[...]
    out_ref[:, 1:1 + N_CTX, :] = ctx_ref[...]
    out_ref[:, 1 + N_CTX:SEQ, :] = suf_ref[:, :SUF, :]
    otok_ref[...] = tokc_ref[...]


def _tc_asm(i, prompts, tokenized, pre, ctx, suf, tok):
    blk = 32
    grid = (2, BC // blk)
    nb, cb = B // blk, BC // blk

    def seg_map(pol, s):
        return (pol * cb + s, 0)

    def seg3_map(pol, s):
        return (pol * cb + s, 0, 0)

    def out_map(pol, s):
        return (pol * nb + i * cb + s, 0, 0)

    def otok_map(pol, s):
        return (pol * nb + i * cb + s, 0)

    seg_specs = [
        pl.BlockSpec((blk, D), seg_map),
        pl.BlockSpec((blk, N_CTX, D), seg3_map),
        pl.BlockSpec((blk, SUF_PAD, D), seg3_map),
        pl.BlockSpec((blk, SEQ_PAD), seg_map),
    ]
    out_shape = (
        jax.ShapeDtypeStruct((2 * B, SEQ, D), jnp.float32),
        jax.ShapeDtypeStruct((2 * B, SEQ_PAD), jnp.int32),
    )
    out_specs = (
        pl.BlockSpec((blk, SEQ, D), out_map),
        pl.BlockSpec((blk, SEQ_PAD), otok_map),
    )
    if prompts is None:
        return pl.pallas_call(
            _tc_asm_body, grid=grid, in_specs=seg_specs,
            out_specs=out_specs, out_shape=out_shape,
        )(pre, ctx, suf, tok)
    return pl.pallas_call(
        _tc_asm_body, grid=grid,
        in_specs=[pl.BlockSpec(memory_space=pl.ANY),
                  pl.BlockSpec(memory_space=pl.ANY)] + seg_specs,
        out_specs=out_specs, out_shape=out_shape,
        input_output_aliases={0: 0, 1: 1},
    )(prompts, tokenized, pre, ctx, suf, tok)


@jax.jit
def _prompt_gather(cls_id, ctx_pos, ctx_neg, pre_pos2, suf_pos,
                   pre_neg2, suf_neg, tail_pos, tail_neg, tok_neg, tok_pos,
                   pre_pos3, pre_neg3):
    del pre_pos3, pre_neg3
    call = _make_call()
    prompts, tokenized = None, None
    for i in range(NCHUNK):
        cls_c = lax.dynamic_slice_in_dim(cls_id, i * BC, BC)
        cls_w = cls_c.reshape(NW, BPW)
        cls2 = jnp.concatenate([cls_w] * 2, axis=1)
        pre, ctx, suf, tok = call(
            cls_w,
            cls2.reshape(NW, 2, BPW),
            cls2.reshape(NW, 2 * BPW // CTX_CH, CTX_CH),
            cls2.reshape(NW, 2 * BPW // TAIL_CH, TAIL_CH),
            pre_neg2, ctx_neg, suf_neg,
            pre_pos2, ctx_pos, suf_pos,
            tail_neg, tail_pos, tok_neg, tok_pos)
        prompts, tokenized = _tc_asm(i, prompts, tokenized,
                                     pre, ctx, suf, tok)
    return prompts, tokenized[:, :SEQ]


def kernel(cls_id, ctx_pos, ctx_neg, token_prefix_pos, token_suffix_pos,
           token_prefix_neg, token_suffix_neg, tokenized_prompts):
    n_cls = ctx_pos.shape[0]
    pad_tail = ((0, 0), (0, SUF_TAIL - (SUF - SUF_MAIN)), (0, 0))
    return _prompt_gather(
        cls_id, ctx_pos, ctx_neg,
        token_prefix_pos.reshape(n_cls, D),
        token_suffix_pos,
        token_prefix_neg.reshape(n_cls, D),
        token_suffix_neg,
        jnp.pad(token_suffix_pos[:, SUF_MAIN:, :], pad_tail),
        jnp.pad(token_suffix_neg[:, SUF_MAIN:, :], pad_tail),
        jnp.pad(tokenized_prompts[:n_cls], ((0, 0), (0, SEQ_PAD - SEQ))),
        jnp.pad(tokenized_prompts[n_cls:], ((0, 0), (0, SEQ_PAD - SEQ))),
        token_prefix_pos, token_prefix_neg,
    )


# speedup vs baseline: 4.1204x; 2.9918x over previous
import jax
import jax.numpy as jnp
from jax import lax
from jax.experimental import pallas as pl
from jax.experimental.pallas import tpu as pltpu
from jax.experimental.pallas import tpu_sc as plsc

N_CLS = 1000
N_CTX = 16
D = 512
SEQ = 77
SUF = SEQ - 1 - N_CTX
B = 1024
SEQ_PAD = 128

NC, NS = 2, 16
NW = NC * NS
BPW = B // NW
NBUF = 4
ROW_BYTES = BPW * D * 4


def _sc_body(cls_w,
             pre_n, ctx_n, suf_n, pre_p, ctx_p, suf_p, tok_n, tok_p,
             out, out_tok,
             idx_v, ixb0, ixb1, ixb2, ixb3,
             gb0, gb1, gb2, gb3, tbuf,
             gs0, gs1, gs2, gs3, ss0, ss1, ss2, ss3, tsem):
    wid = lax.axis_index("s") * NC + lax.axis_index("c")
    base = wid * BPW

    pltpu.sync_copy(cls_w.at[wid], idx_v)

    ixb = [ixb0, ixb1, ixb2, ixb3]
    gb = [gb0, gb1, gb2, gb3]
    gs = [gs0, gs1, gs2, gs3]
    ss = [ss0, ss1, ss2, ss3]

    state = {"t": 0}

    def drain_scatter(b):
        pltpu.make_async_copy(gb[b], out.at[pl.ds(0, BPW), :], ss[b]).wait()

    def set_idx(b, mul, off):
        for h in range(BPW // 16):
            v = idx_v[pl.ds(h * 16, 16)]
            ixb[b][pl.ds(h * 16, 16)] = v * mul + off

    def task(table, mul, off, row0):
        b = state["t"] % NBUF
        if state["t"] >= NBUF:
            drain_scatter(b)
        state["t"] += 1
        set_idx(b, mul, off)
        pltpu.async_copy(table.at[ixb[b]], gb[b], gs[b]).wait()
        pltpu.async_copy(gb[b], out.at[pl.ds(row0, BPW), :], ss[b])

    for pol, (pre_t, ctx_t, suf_t, tok_t) in enumerate(
            ((pre_n, ctx_n, suf_n, tok_n), (pre_p, ctx_p, suf_p, tok_p))):
        ob = pol * B + base
        task(pre_t, 1, 0, ob)
        for k in range(N_CTX):
            task(ctx_t, N_CTX, k, (1 + k) * 2 * B + ob)

        t0 = state["t"]

        def suf_step(g, _, suf_t=suf_t, ob=ob, t0=t0):
            for bb in range(NBUF):
                s = g * NBUF + bb
                b = (t0 + bb) % NBUF
                drain_scatter(b)
                set_idx(b, 1, s * N_CLS)
                pltpu.async_copy(suf_t.at[ixb[b]], gb[b], gs[b]).wait()
                pltpu.async_copy(
                    gb[b],
                    out.at[pl.ds((1 + N_CTX + s) * 2 * B + ob, BPW), :],
                    ss[b])
            return ()

        lax.fori_loop(0, SUF // NBUF, suf_step, ())
        state["t"] += SUF

        pltpu.async_copy(tok_t.at[idx_v], tbuf, tsem).wait()
        pltpu.sync_copy(tbuf, out_tok.at[pl.ds(ob, BPW), :])

    for b in range(NBUF):
        drain_scatter(b)


def _make_call():
    mesh = plsc.VectorSubcoreMesh(core_axis_name="c", subcore_axis_name="s",
                                  num_cores=NC, num_subcores=NS)
    return pl.kernel(
        _sc_body,
        out_type=(
            jax.ShapeDtypeStruct((SEQ * 2 * B, D), jnp.float32),
            jax.ShapeDtypeStruct((2 * B, SEQ_PAD), jnp.int32),
        ),
        mesh=mesh,
        scratch_types=[
            pltpu.VMEM((BPW,), jnp.int32),
            pltpu.VMEM((BPW,), jnp.int32),
            pltpu.VMEM((BPW,), jnp.int32),
            pltpu.VMEM((BPW,), jnp.int32),
            pltpu.VMEM((BPW,), jnp.int32),
            pltpu.VMEM((BPW, D), jnp.float32),
            pltpu.VMEM((BPW, D), jnp.float32),
            pltpu.VMEM((BPW, D), jnp.float32),
            pltpu.VMEM((BPW, D), jnp.float32),
            pltpu.VMEM((BPW, SEQ_PAD), jnp.int32),
        ] + [pltpu.SemaphoreType.DMA] * 9,
    )


@jax.jit
def _prompt_gather(cls_id, ctx_pos, ctx_neg, token_prefix_pos,
                   token_suffix_pos, token_prefix_neg, token_suffix_neg,
                   tok_neg, tok_pos):
    n_cls = ctx_pos.shape[0]
    cls_w = cls_id.reshape(NW, BPW)
    suf_pos2 = jnp.transpose(token_suffix_pos, (1, 0, 2)).reshape(
        SUF * n_cls, D)
    suf_neg2 = jnp.transpose(token_suffix_neg, (1, 0, 2)).reshape(
        SUF * n_cls, D)
    ctx_pos2 = ctx_pos.reshape(n_cls * N_CTX, D)
    ctx_neg2 = ctx_neg.reshape(n_cls * N_CTX, D)
    pre_pos2 = token_prefix_pos.reshape(n_cls, D)
    pre_neg2 = token_prefix_neg.reshape(n_cls, D)
    call = _make_call()
    out2, out_tok = call(cls_w,
                         pre_neg2, ctx_neg2, suf_neg2,
                         pre_pos2, ctx_pos2, suf_pos2,
                         tok_neg, tok_pos)
    prompts = jnp.transpose(out2.reshape(SEQ, 2 * B, D), (1, 0, 2))
    return prompts, out_tok[:, :SEQ]


def kernel(cls_id, ctx_pos, ctx_neg, token_prefix_pos, token_suffix_pos,
           token_prefix_neg, token_suffix_neg, tokenized_prompts):
    n_cls = ctx_pos.shape[0]
    return _prompt_gather(
        cls_id, ctx_pos, ctx_neg,
        token_prefix_pos, token_suffix_pos,
        token_prefix_neg, token_suffix_neg,
        jnp.pad(tokenized_prompts[:n_cls], ((0, 0), (0, SEQ_PAD - SEQ))),
        jnp.pad(tokenized_prompts[n_cls:], ((0, 0), (0, SEQ_PAD - SEQ))),
    )


# speedup vs baseline: 5.1175x; 1.2420x over previous
import jax
import jax.numpy as jnp
from jax import lax
from jax.experimental import pallas as pl
from jax.experimental.pallas import tpu as pltpu
from jax.experimental.pallas import tpu_sc as plsc

N_CLS = 1000
N_CTX = 16
D = 512
SEQ = 77
SUF = SEQ - 1 - N_CTX
B = 1024
SEQ_PAD = 128

NC, NS = 2, 16
NW = NC * NS
BPW = B // NW
NBUF = 4
ROW_BYTES = BPW * D * 4


def _sc_body(cls_w,
             pre_n, ctx_n, suf_n, pre_p, ctx_p, suf_p, tok_n, tok_p,
             out, out_tok,
             idx_v, ixb0, ixb1, ixb2, ixb3,
             gb0, gb1, gb2, gb3, tbuf,
             gs0, gs1, gs2, gs3, ss0, ss1, ss2, ss3, tsem):
    wid = lax.axis_index("s") * NC + lax.axis_index("c")
    base = wid * BPW

    pltpu.sync_copy(cls_w.at[wid], idx_v)

    ixb = [ixb0, ixb1, ixb2, ixb3]
    gb = [gb0, gb1, gb2, gb3]
    gs = [gs0, gs1, gs2, gs3]
    ss = [ss0, ss1, ss2, ss3]

    state = {"t": 0}

    def drain_scatter(b):
        pltpu.make_async_copy(gb[b], out.at[pl.ds(0, BPW), :], ss[b]).wait()

    def wait_gather(b):
        pltpu.make_async_copy(out.at[pl.ds(0, BPW), :], gb[b], gs[b]).wait()

    def set_idx(b, mul, off):
        for h in range(BPW // 16):
            v = idx_v[pl.ds(h * 16, 16)]
            ixb[b][pl.ds(h * 16, 16)] = v * mul + off

    def group(tasks):
        bs = []
        for table, mul, off, row0 in tasks:
            b = state["t"] % NBUF
            if state["t"] >= NBUF:
                drain_scatter(b)
            state["t"] += 1
            set_idx(b, mul, off)
            pltpu.async_copy(table.at[ixb[b]], gb[b], gs[b])
            bs.append((b, row0))
        for b, row0 in bs:
            wait_gather(b)
            pltpu.async_copy(gb[b], out.at[pl.ds(row0, BPW), :], ss[b])

    for pol, (pre_t, ctx_t, suf_t, tok_t) in enumerate(
            ((pre_n, ctx_n, suf_n, tok_n), (pre_p, ctx_p, suf_p, tok_p))):
        ob = pol * B + base
        static_tasks = [(pre_t, 1, 0, ob)]
        for k in range(N_CTX):
            static_tasks.append((ctx_t, N_CTX, k, (1 + k) * 2 * B + ob))
        for g0 in range(0, len(static_tasks), NBUF):
            group(static_tasks[g0:g0 + NBUF])

        t0 = state["t"]

        def suf_step(g, _, suf_t=suf_t, ob=ob, t0=t0):
            bs = []
            for bb in range(NBUF):
                s = g * NBUF + bb
                b = (t0 + bb) % NBUF
                drain_scatter(b)
                set_idx(b, 1, s * N_CLS)
                pltpu.async_copy(suf_t.at[ixb[b]], gb[b], gs[b])
                bs.append((b, s))
            for b, s in bs:
                wait_gather(b)
                pltpu.async_copy(
                    gb[b],
                    out.at[pl.ds((1 + N_CTX + s) * 2 * B + ob, BPW), :],
                    ss[b])
            return ()

        lax.fori_loop(0, SUF // NBUF, suf_step, ())
        state["t"] += SUF

        pltpu.async_copy(tok_t.at[idx_v], tbuf, tsem).wait()
        pltpu.sync_copy(tbuf, out_tok.at[pl.ds(ob, BPW), :])

    for b in range(NBUF):
        drain_scatter(b)


def _make_call():
    mesh = plsc.VectorSubcoreMesh(core_axis_name="c", subcore_axis_name="s",
                                  num_cores=NC, num_subcores=NS)
    return pl.kernel(
        _sc_body,
        out_type=(
            jax.ShapeDtypeStruct((SEQ * 2 * B, D), jnp.float32),
            jax.ShapeDtypeStruct((2 * B, SEQ_PAD), jnp.int32),
        ),
        mesh=mesh,
        scratch_types=[
            pltpu.VMEM((BPW,), jnp.int32),
            pltpu.VMEM((BPW,), jnp.int32),
            pltpu.VMEM((BPW,), jnp.int32),
            pltpu.VMEM((BPW,), jnp.int32),
            pltpu.VMEM((BPW,), jnp.int32),
            pltpu.VMEM((BPW, D), jnp.float32),
            pltpu.VMEM((BPW, D), jnp.float32),
            pltpu.VMEM((BPW, D), jnp.float32),
            pltpu.VMEM((BPW, D), jnp.float32),
            pltpu.VMEM((BPW, SEQ_PAD), jnp.int32),
        ] + [pltpu.SemaphoreType.DMA] * 9,
    )


@jax.jit
def _prompt_gather(cls_id, ctx_pos, ctx_neg, token_prefix_pos,
                   token_suffix_pos, token_prefix_neg, token_suffix_neg,
                   tok_neg, tok_pos):
    n_cls = ctx_pos.shape[0]
    cls_w = cls_id.reshape(NW, BPW)
    suf_pos2 = jnp.transpose(token_suffix_pos, (1, 0, 2)).reshape(
        SUF * n_cls, D)
    suf_neg2 = jnp.transpose(token_suffix_neg, (1, 0, 2)).reshape(
        SUF * n_cls, D)
    ctx_pos2 = ctx_pos.reshape(n_cls * N_CTX, D)
    ctx_neg2 = ctx_neg.reshape(n_cls * N_CTX, D)
    pre_pos2 = token_prefix_pos.reshape(n_cls, D)
    pre_neg2 = token_prefix_neg.reshape(n_cls, D)
    call = _make_call()
    out2, out_tok = call(cls_w,
                         pre_neg2, ctx_neg2, suf_neg2,
                         pre_pos2, ctx_pos2, suf_pos2,
                         tok_neg, tok_pos)
    prompts = jnp.transpose(out2.reshape(SEQ, 2 * B, D), (1, 0, 2))
    return prompts, out_tok[:, :SEQ]


def kernel(cls_id, ctx_pos, ctx_neg, token_prefix_pos, token_suffix_pos,
           token_prefix_neg, token_suffix_neg, tokenized_prompts):
    n_cls = ctx_pos.shape[0]
    return _prompt_gather(
        cls_id, ctx_pos, ctx_neg,
        token_prefix_pos, token_suffix_pos,
        token_prefix_neg, token_suffix_neg,
        jnp.pad(tokenized_prompts[:n_cls], ((0, 0), (0, SEQ_PAD - SEQ))),
        jnp.pad(tokenized_prompts[n_cls:], ((0, 0), (0, SEQ_PAD - SEQ))),
    )
